# baseline (speedup 1.0000x reference)
import functools

import jax
import jax.numpy as jnp
from jax import lax
from jax.experimental import pallas as pl
from jax.experimental.pallas import tpu as pltpu
from jax.experimental.pallas import tpu_sc as plsc

N_NODES = 10000
E_EDGES = 320000
H = 8
D = 16
HD = H * D
B_TGT = 1024

NC = 2
NS = 16
L = 16
NW = NC * NS
EPW = E_EDGES // NW
EB = 80
NBLK = EPW // EB
NG = EB // L
CW = 64
WLR = 3
MR = EB + CW
TRASH = B_TGT
ACC_ROWS = 1152
TPW = B_TGT // NW


def _mesh():
    return plsc.VectorSubcoreMesh(
        core_axis_name="c", subcore_axis_name="s",
        num_cores=NC, num_subcores=NS)


def _build_map(tgt_hbm, tgtb, nmap):
    pltpu.sync_copy(tgt_hbm, tgtb.at[pl.ds(0, B_TGT)])
    neg = jnp.full((L,), -1, jnp.int32)
    lane0 = lax.iota(jnp.int32, L) == 0

    def zbody(i, c):
        nmap[pl.ds(i * L, L)] = neg
        return c
    lax.fori_loop(0, N_NODES // L + 1, zbody, 0)

    def mbody(b, c):
        tv = tgtb[pl.ds(b, L)]
        plsc.store_scatter(nmap, [tv], jnp.full((L,), b, jnp.int32),
                           mask=lane0)
        return c
    lax.fori_loop(0, B_TGT, mbody, 0)


@functools.partial(
    pl.kernel,
    out_type=(
        jax.ShapeDtypeStruct((E_EDGES * H,), jnp.float32),
        jax.ShapeDtypeStruct((NC, B_TGT, HD), jnp.float32),
    ),
    mesh=_mesh(),
    scratch_types=[
        pltpu.VMEM_SHARED((ACC_ROWS, HD), jnp.float32),
        pltpu.VMEM((N_NODES + L,), jnp.int32),
        pltpu.VMEM((B_TGT + L,), jnp.int32),
        pltpu.VMEM((2 * EB * HD,), jnp.float32),
        pltpu.VMEM((2 * MR, HD), jnp.float32),
        pltpu.VMEM((EB * H,), jnp.float32),
        pltpu.VMEM((2 * EB,), jnp.int32),
        pltpu.VMEM((EB,), jnp.int32),
        pltpu.VMEM((WLR, CW), jnp.int32),
        pltpu.SemaphoreType.DMA,
        pltpu.SemaphoreType.DMA,
        pltpu.SemaphoreType.DMA,
    ],
    compiler_params=pltpu.CompilerParams(needs_layout_passes=False),
)
def _phase1(dst_hbm, node_hbm, eft_hbm, tgt_hbm, a_hbm, part_hbm,
            acc, nmap, tgtb, eftv, nrow, av, dstv, hid, wl,
            sem_d, sem_e, sem_g):
    c = lax.axis_index("c")
    s = lax.axis_index("s")
    zero16 = jnp.zeros((L,), jnp.float32)
    zero16i = jnp.zeros((L,), jnp.int32)
    iota = lax.iota(jnp.int32, L)
    tile_base = c * (NS * EPW) + s * EPW

    _build_map(tgt_hbm, tgtb, nmap)

    def hz(i, carry):
        hid[pl.ds(i * L, L)] = zero16i
        return carry
    lax.fori_loop(0, EB // L, hz, 0)

    def zrow(i, carry):
        for j in range(H):
            nrow[i, pl.ds(j * L, L)] = zero16
        return carry
    lax.fori_loop(0, ACC_ROWS // NS, zrow, 0)
    pltpu.sync_copy(nrow.at[pl.ds(0, ACC_ROWS // NS)],
                    acc.at[pl.ds(s * (ACC_ROWS // NS), ACC_ROWS // NS)])
    plsc.subcore_barrier()

    def _eft_slice(b):
        return eft_hbm.at[pl.ds((tile_base + b * EB) * HD, EB * HD)]

    def _dst_slice(b):
        return dst_hbm.at[pl.ds(tile_base + b * EB, EB)]

    pltpu.sync_copy(_dst_slice(0), dstv.at[pl.ds(0, EB)])
    pltpu.async_copy(_eft_slice(0), eftv.at[pl.ds(0, EB * HD)], sem_e)
    pltpu.async_copy(node_hbm.at[dstv.at[pl.ds(0, EB)]],
                     nrow.at[pl.ds(0, EB)], sem_g)

    def block_body(blk, carry):
        p = blk & 1
        q = 1 - p
        base = tile_base + blk * EB
        nb = blk + 1

        eoff = p * (EB * HD)
        roff = p * MR
        doff = p * EB

        @pl.when(nb < NBLK)
        def _():
            pltpu.async_copy(_dst_slice(nb),
                             dstv.at[pl.ds(q * EB, EB)], sem_d)

        pltpu.make_async_copy(_eft_slice(blk),
                              eftv.at[pl.ds(eoff, EB * HD)], sem_e).wait()
        pltpu.make_async_copy(node_hbm.at[dstv.at[pl.ds(doff, EB)]],
                              nrow.at[pl.ds(roff, EB)], sem_g).wait()

        def group_body(g, cnt):
            rows = g * L + iota
            rowsr = rows + roff
            ebase = rows * HD + eoff
            acc_a = [None] * H
            acc_b = [None] * H
            for d in range(D):
                ed = ebase + d
                for h in range(H):
                    col = h * D + d
                    et = plsc.load_gather(eftv, [ed + h * D])
                    nt = plsc.load_gather(
                        nrow, [rowsr, jnp.full((L,), col, jnp.int32)])
                    pr = et * nt
                    if d % 2 == 0:
                        acc_a[h] = pr if acc_a[h] is None else acc_a[h] + pr
                    else:
                        acc_b[h] = pr if acc_b[h] is None else acc_b[h] + pr
            sims = [acc_a[h] + acc_b[h] for h in range(H)]
            m = sims[0]
            for h in range(1, H):
                m = jnp.maximum(m, sims[h])
            zs = [jnp.exp(sims[h] - m) for h in range(H)]
            ssum = (((zs[0] + zs[1]) + (zs[2] + zs[3]))
                    + ((zs[4] + zs[5]) + (zs[6] + zs[7])))
            rs = 1.0 / ssum
            a_hs = [zs[h] * rs for h in range(H)]
            abase = rows * H
            for h in range(H):
                plsc.store_scatter(av, [abase + h], a_hs[h])
            dvec = dstv[pl.ds(doff + g * L, L)]
            slots = plsc.load_gather(nmap, [dvec])
            hit = slots >= 0
            hi = hit.astype(jnp.int32)
            pos = cnt + plsc.cumsum(hi) - hi
            plsc.store_scatter(hid, [pos], rows, mask=hit)
            plsc.store_scatter(wl, [pos >> 6, pos & (CW - 1)], slots,
                               mask=hit)
            return cnt + jnp.sum(hi)
        cnt = lax.fori_loop(0, NG, group_body, jnp.int32(0))

        trash = jnp.full((L,), TRASH, jnp.int32)
        for t in range(CW // L):
            pp = cnt + t * L + iota
            plsc.store_scatter(wl, [pp >> 6, pp & (CW - 1)], trash)

        def hit_body(k, carry):
            drow = k * L + iota
            eids = plsc.load_gather(hid, [drow])
            ebase = eids * HD + eoff
            a_hs = [plsc.load_gather(av, [eids * H + h]) for h in range(H)]
            for h in range(H):
                for d in range(D):
                    col = h * D + d
                    et = plsc.load_gather(eftv, [ebase + col])
                    plsc.store_scatter(
                        nrow, [drow + roff, jnp.full((L,), col, jnp.int32)],
                        et * a_hs[h])
            return carry
        lax.fori_loop(0, (cnt + L - 1) >> 4, hit_body, 0)

        pltpu.sync_copy(av, a_hbm.at[pl.ds(base * H, EB * H)])
        nchunk = (cnt + CW - 1) >> 6

        def sc_body(j, carry):
            pltpu.sync_copy(nrow.at[pl.ds(roff + j * CW, CW)],
                            acc.at[wl.at[j]], add=True)
            return carry
        lax.fori_loop(0, nchunk, sc_body, 0)

        @pl.when(nb < NBLK)
        def _():
            pltpu.make_async_copy(_dst_slice(nb),
                                  dstv.at[pl.ds(q * EB, EB)], sem_d).wait()
            pltpu.async_copy(_eft_slice(nb),
                             eftv.at[pl.ds(q * EB * HD, EB * HD)], sem_e)
            pltpu.async_copy(node_hbm.at[dstv.at[pl.ds(q * EB, EB)]],
                             nrow.at[pl.ds(q * MR, EB)], sem_g)
        return 0
    lax.fori_loop(0, NBLK, block_body, 0)

    plsc.subcore_barrier()
    r0 = s * (B_TGT // NS)
    pltpu.sync_copy(acc.at[pl.ds(r0, B_TGT // NS)],
                    nrow.at[pl.ds(0, B_TGT // NS)])
    pltpu.sync_copy(nrow.at[pl.ds(0, B_TGT // NS)],
                    part_hbm.at[c].at[pl.ds(r0, B_TGT // NS)])


@functools.partial(
    pl.kernel,
    out_type=jax.ShapeDtypeStruct((B_TGT, HD), jnp.float32),
    mesh=_mesh(),
    scratch_types=[
        pltpu.VMEM((N_NODES + L,), jnp.int32),
        pltpu.VMEM((B_TGT + L,), jnp.int32),
        pltpu.VMEM((TPW,), jnp.int32),
        pltpu.VMEM((TPW, HD), jnp.float32),
        pltpu.VMEM((TPW, HD), jnp.float32),
        pltpu.VMEM((TPW, HD), jnp.float32),
        pltpu.SemaphoreType.DMA,
    ],
    compiler_params=pltpu.CompilerParams(needs_layout_passes=False),
)
def _phase2(p0_hbm, p1_hbm, tgt_hbm, out_hbm,
            nmap, tgtb, slotv, r0v, r1v, ov, sem):
    c = lax.axis_index("c")
    s = lax.axis_index("s")
    base = (s * NC + c) * TPW

    _build_map(tgt_hbm, tgtb, nmap)
    for k in range(TPW // L):
        tv = tgtb[pl.ds(base + k * L, L)]
        slotv[pl.ds(k * L, L)] = plsc.load_gather(nmap, [tv])
    pltpu.async_copy(p0_hbm.at[slotv], r0v, sem).wait()
    pltpu.async_copy(p1_hbm.at[slotv], r1v, sem).wait()

    def row_body(i, carry):
        vs = []
        for j in range(H):
            vs.append(r0v[i, pl.ds(j * D, D)] + r1v[i, pl.ds(j * D, D)]
                      + 1e-15)
        ssq = vs[0] * vs[0]
        for j in range(1, H):
            ssq = ssq + vs[j] * vs[j]
        ib = plsc.bitcast(ssq, jnp.int32)
        y = plsc.bitcast(jnp.int32(0x5F3759DF) - (ib >> 1), jnp.float32)
        for _ in range(3):
            y = y * (1.5 - 0.5 * ssq * y * y)
        y = jnp.minimum(y, 1e12)
        for j in range(H):
            ov[i, pl.ds(j * D, D)] = vs[j] * y
        return carry
    lax.fori_loop(0, TPW, row_body, 0)
    pltpu.sync_copy(ov, out_hbm.at[pl.ds(base, TPW)])


def kernel(edge_index, node, eft, target_idx):
    dst = edge_index[1]
    node2 = node.reshape(N_NODES, HD)
    eftf = eft.reshape(E_EDGES * HD)
    a_flat, parts = _phase1(dst, node2, eftf, target_idx)
    out2 = _phase2(parts[0], parts[1], target_idx)
    return (out2.reshape(B_TGT, H, D), a_flat.reshape(E_EDGES, H, 1))


# speedup vs baseline: 1.8114x; 1.8114x over previous
import functools

import jax
import jax.numpy as jnp
from jax import lax
from jax.experimental import pallas as pl
from jax.experimental.pallas import tpu as pltpu
from jax.experimental.pallas import tpu_sc as plsc

N_NODES = 10000
E_EDGES = 320000
H = 8
D = 16
HD = H * D
B_TGT = 1024

NC = 2
NS = 16
L = 16
NW = NC * NS
EPW = E_EDGES // NW
EB = 80
NBLK = EPW // EB
NG = EB // L
CW = 64
WLR = 3
MR = EB + CW
TRASH = B_TGT
ACC_ROWS = 1152
TPW = B_TGT // NW


def _mesh():
    return plsc.VectorSubcoreMesh(
        core_axis_name="c", subcore_axis_name="s",
        num_cores=NC, num_subcores=NS)


def _build_map(tgt_hbm, tgtb, nmap):
    pltpu.sync_copy(tgt_hbm, tgtb.at[pl.ds(0, B_TGT)])
    neg = jnp.full((L,), -1, jnp.int32)
    lane0 = lax.iota(jnp.int32, L) == 0

    def zbody(i, c):
        nmap[pl.ds(i * L, L)] = neg
        return c
    lax.fori_loop(0, N_NODES // L + 1, zbody, 0)

    def mbody(b, c):
        tv = tgtb[pl.ds(b, L)]
        plsc.store_scatter(nmap, [tv], jnp.full((L,), b, jnp.int32),
                           mask=lane0)
        return c
    lax.fori_loop(0, B_TGT, mbody, 0)


@functools.partial(
    pl.kernel,
    out_type=(
        jax.ShapeDtypeStruct((E_EDGES * H,), jnp.float32),
        jax.ShapeDtypeStruct((NC, B_TGT, HD), jnp.float32),
    ),
    mesh=_mesh(),
    scratch_types=[
        pltpu.VMEM_SHARED((ACC_ROWS, HD), jnp.float32),
        pltpu.VMEM((N_NODES + L,), jnp.int32),
        pltpu.VMEM((B_TGT + L,), jnp.int32),
        pltpu.VMEM((2 * EB * HD,), jnp.float32),
        pltpu.VMEM((2 * EB, HD), jnp.float32),
        pltpu.VMEM((MR, HD), jnp.float32),
        pltpu.VMEM((EB * H,), jnp.float32),
        pltpu.VMEM((2 * EB,), jnp.int32),
        pltpu.VMEM((EB,), jnp.int32),
        pltpu.VMEM((WLR, CW), jnp.int32),
        pltpu.SemaphoreType.DMA,
        pltpu.SemaphoreType.DMA,
        pltpu.SemaphoreType.DMA,
    ],
    compiler_params=pltpu.CompilerParams(needs_layout_passes=False),
)
def _phase1(dst_hbm, node_hbm, eft_hbm, tgt_hbm, a_hbm, part_hbm,
            acc, nmap, tgtb, eftv, nob, msg, av, dstv, hid, wl,
            sem_d, sem_e, sem_g):
    c = lax.axis_index("c")
    s = lax.axis_index("s")
    zero16 = jnp.zeros((L,), jnp.float32)
    zero16i = jnp.zeros((L,), jnp.int32)
    iota = lax.iota(jnp.int32, L)
    tile_base = c * (NS * EPW) + s * EPW

    _build_map(tgt_hbm, tgtb, nmap)

    def hz(i, carry):
        hid[pl.ds(i * L, L)] = zero16i
        return carry
    lax.fori_loop(0, EB // L, hz, 0)

    def zrow(i, carry):
        for j in range(H):
            msg[i, pl.ds(j * L, L)] = zero16
        return carry
    lax.fori_loop(0, ACC_ROWS // NS, zrow, 0)
    pltpu.sync_copy(msg.at[pl.ds(0, ACC_ROWS // NS)],
                    acc.at[pl.ds(s * (ACC_ROWS // NS), ACC_ROWS // NS)])
    plsc.subcore_barrier()

    def _eft_slice(b):
        return eft_hbm.at[pl.ds((tile_base + b * EB) * HD, EB * HD)]

    def _dst_slice(b):
        return dst_hbm.at[pl.ds(tile_base + b * EB, EB)]

    pltpu.sync_copy(_dst_slice(0), dstv.at[pl.ds(0, EB)])
    pltpu.async_copy(_eft_slice(0), eftv.at[pl.ds(0, EB * HD)], sem_e)
    pltpu.async_copy(node_hbm.at[dstv.at[pl.ds(0, EB)]],
                     nob.at[pl.ds(0, EB)], sem_g)

    def block_body(blk, carry):
        p = blk & 1
        q = 1 - p
        base = tile_base + blk * EB
        nb = blk + 1

        eoff = p * (EB * HD)
        noff = p * EB
        doff = p * EB

        @pl.when(nb < NBLK)
        def _():
            pltpu.async_copy(_dst_slice(nb),
                             dstv.at[pl.ds(q * EB, EB)], sem_d)

        pltpu.make_async_copy(_eft_slice(blk),
                              eftv.at[pl.ds(eoff, EB * HD)], sem_e).wait()
        pltpu.make_async_copy(node_hbm.at[dstv.at[pl.ds(doff, EB)]],
                              nob.at[pl.ds(noff, EB)], sem_g).wait()

        def group_body(g, cnt):
            rows = g * L + iota
            ebase = rows * HD + eoff
            rowsn = rows + noff

            def dot_body(t, carry):
                iotat, sims = carry
                col = iotat & (D - 1)
                new = []
                for h in range(H):
                    et = plsc.load_gather(eftv, [ebase + col])
                    nt = plsc.load_gather(nob, [rowsn, col])
                    new.append(sims[h] + et * nt)
                    if h < H - 1:
                        col = col + D
                return (iotat + 1, tuple(new))
            _, sims = lax.fori_loop(
                0, D, dot_body,
                (iota, tuple(jnp.zeros((L,), jnp.float32)
                             for _ in range(H))))
            sims = list(sims)
            m = sims[0]
            for h in range(1, H):
                m = jnp.maximum(m, sims[h])
            zs = [jnp.exp(sims[h] - m) for h in range(H)]
            ssum = (((zs[0] + zs[1]) + (zs[2] + zs[3]))
                    + ((zs[4] + zs[5]) + (zs[6] + zs[7])))
            rs = 1.0 / ssum
            a_hs = [zs[h] * rs for h in range(H)]
            abase = rows * H
            for h in range(H):
                plsc.store_scatter(av, [abase + h], a_hs[h])
            dvec = dstv[pl.ds(doff + g * L, L)]
            slots = plsc.load_gather(nmap, [dvec])
            hit = slots >= 0
            hi = hit.astype(jnp.int32)
            pos = cnt + plsc.cumsum(hi) - hi
            plsc.store_scatter(hid, [pos], rows, mask=hit)
            plsc.store_scatter(wl, [pos >> 6, pos & (CW - 1)], slots,
                               mask=hit)
            return cnt + jnp.sum(hi)
        cnt = lax.fori_loop(0, NG, group_body, jnp.int32(0))

        trash = jnp.full((L,), TRASH, jnp.int32)
        for t in range(CW // L):
            pp = cnt + t * L + iota
            plsc.store_scatter(wl, [pp >> 6, pp & (CW - 1)], trash)

        def hit_body(k, carry):
            drow = k * L + iota
            eids = plsc.load_gather(hid, [drow])
            ebase = eids * HD + eoff
            a_hs = [plsc.load_gather(av, [eids * H + h]) for h in range(H)]

            def msg_body(t, mc):
                iotat = mc
                col = iotat & (D - 1)
                for h in range(H):
                    et = plsc.load_gather(eftv, [ebase + col])
                    plsc.store_scatter(msg, [drow, col], et * a_hs[h])
                    if h < H - 1:
                        col = col + D
                return iotat + 1
            lax.fori_loop(0, D, msg_body, iota)
            return carry
        lax.fori_loop(0, (cnt + L - 1) >> 4, hit_body, 0)

        pltpu.sync_copy(av, a_hbm.at[pl.ds(base * H, EB * H)])
        nchunk = (cnt + CW - 1) >> 6

        def sc_body(j, carry):
            pltpu.sync_copy(msg.at[pl.ds(j * CW, CW)],
                            acc.at[wl.at[j]], add=True)
            return carry
        lax.fori_loop(0, nchunk, sc_body, 0)

        @pl.when(nb < NBLK)
        def _():
            pltpu.make_async_copy(_dst_slice(nb),
                                  dstv.at[pl.ds(q * EB, EB)], sem_d).wait()
            pltpu.async_copy(_eft_slice(nb),
                             eftv.at[pl.ds(q * EB * HD, EB * HD)], sem_e)
            pltpu.async_copy(node_hbm.at[dstv.at[pl.ds(q * EB, EB)]],
                             nob.at[pl.ds(q * EB, EB)], sem_g)
        return 0
    lax.fori_loop(0, NBLK, block_body, 0)

    plsc.subcore_barrier()
    r0 = s * (B_TGT // NS)
    pltpu.sync_copy(acc.at[pl.ds(r0, B_TGT // NS)],
                    msg.at[pl.ds(0, B_TGT // NS)])
    pltpu.sync_copy(msg.at[pl.ds(0, B_TGT // NS)],
                    part_hbm.at[c].at[pl.ds(r0, B_TGT // NS)])


@functools.partial(
    pl.kernel,
    out_type=jax.ShapeDtypeStruct((B_TGT, HD), jnp.float32),
    mesh=_mesh(),
    scratch_types=[
        pltpu.VMEM((N_NODES + L,), jnp.int32),
        pltpu.VMEM((B_TGT + L,), jnp.int32),
        pltpu.VMEM((TPW,), jnp.int32),
        pltpu.VMEM((TPW, HD), jnp.float32),
        pltpu.VMEM((TPW, HD), jnp.float32),
        pltpu.VMEM((TPW, HD), jnp.float32),
        pltpu.SemaphoreType.DMA,
    ],
    compiler_params=pltpu.CompilerParams(needs_layout_passes=False),
)
def _phase2(p0_hbm, p1_hbm, tgt_hbm, out_hbm,
            nmap, tgtb, slotv, r0v, r1v, ov, sem):
    c = lax.axis_index("c")
    s = lax.axis_index("s")
    base = (s * NC + c) * TPW

    _build_map(tgt_hbm, tgtb, nmap)
    for k in range(TPW // L):
        tv = tgtb[pl.ds(base + k * L, L)]
        slotv[pl.ds(k * L, L)] = plsc.load_gather(nmap, [tv])
    pltpu.async_copy(p0_hbm.at[slotv], r0v, sem).wait()
    pltpu.async_copy(p1_hbm.at[slotv], r1v, sem).wait()

    def row_body(i, carry):
        vs = []
        for j in range(H):
            vs.append(r0v[i, pl.ds(j * D, D)] + r1v[i, pl.ds(j * D, D)]
                      + 1e-15)
        ssq = vs[0] * vs[0]
        for j in range(1, H):
            ssq = ssq + vs[j] * vs[j]
        ib = plsc.bitcast(ssq, jnp.int32)
        y = plsc.bitcast(jnp.int32(0x5F3759DF) - (ib >> 1), jnp.float32)
        for _ in range(3):
            y = y * (1.5 - 0.5 * ssq * y * y)
        y = jnp.minimum(y, 1e12)
        for j in range(H):
            ov[i, pl.ds(j * D, D)] = vs[j] * y
        return carry
    lax.fori_loop(0, TPW, row_body, 0)
    pltpu.sync_copy(ov, out_hbm.at[pl.ds(base, TPW)])


def kernel(edge_index, node, eft, target_idx):
    dst = edge_index[1]
    node2 = node.reshape(N_NODES, HD)
    eftf = eft.reshape(E_EDGES * HD)
    a_flat, parts = _phase1(dst, node2, eftf, target_idx)
    out2 = _phase2(parts[0], parts[1], target_idx)
    return (out2.reshape(B_TGT, H, D), a_flat.reshape(E_EDGES, H, 1))


# speedup vs baseline: 3.6821x; 2.0327x over previous
import functools

import jax
import jax.numpy as jnp
from jax import lax
from jax.experimental import pallas as pl
from jax.experimental.pallas import tpu as pltpu
from jax.experimental.pallas import tpu_sc as plsc

N_NODES = 10000
E_EDGES = 320000
H = 8
D = 16
HD = H * D
B_TGT = 1024

NC = 2
NS = 16
L = 16
NW = NC * NS
EPW = E_EDGES // NW
EB = 80
NBLK = EPW // EB
NG = EB // L
CW = 64
WLR = 3
MR = EB + CW
TRASH = B_TGT
ACC_ROWS = 1152
TPW = B_TGT // NW


def _mesh():
    return plsc.VectorSubcoreMesh(
        core_axis_name="c", subcore_axis_name="s",
        num_cores=NC, num_subcores=NS)


def _build_map(tgt_hbm, tgtb, nmap):
    pltpu.sync_copy(tgt_hbm, tgtb.at[pl.ds(0, B_TGT)])
    neg = jnp.full((L,), -1, jnp.int32)
    lane0 = lax.iota(jnp.int32, L) == 0

    def zbody(i, c):
        nmap[pl.ds(i * L, L)] = neg
        return c
    lax.fori_loop(0, N_NODES // L + 1, zbody, 0)

    def mbody(b, c):
        tv = tgtb[pl.ds(b, L)]
        plsc.store_scatter(nmap, [tv], jnp.full((L,), b, jnp.int32),
                           mask=lane0)
        return c
    lax.fori_loop(0, B_TGT, mbody, 0)


@functools.partial(
    pl.kernel,
    out_type=(
        jax.ShapeDtypeStruct((E_EDGES, H), jnp.float32),
        jax.ShapeDtypeStruct((NC, B_TGT, HD), jnp.float32),
    ),
    mesh=_mesh(),
    scratch_types=[
        pltpu.VMEM_SHARED((ACC_ROWS, HD), jnp.float32),
        pltpu.VMEM((N_NODES + L,), jnp.int32),
        pltpu.VMEM((B_TGT + L,), jnp.int32),
        pltpu.VMEM((2 * EB, HD), jnp.float32),
        pltpu.VMEM((2 * EB, HD), jnp.float32),
        pltpu.VMEM((MR, HD), jnp.float32),
        pltpu.VMEM((EB, H), jnp.float32),
        pltpu.VMEM((2 * EB,), jnp.int32),
        pltpu.VMEM((EB,), jnp.int32),
        pltpu.VMEM((WLR, CW), jnp.int32),
        pltpu.SemaphoreType.DMA,
        pltpu.SemaphoreType.DMA,
        pltpu.SemaphoreType.DMA,
    ],
    compiler_params=pltpu.CompilerParams(needs_layout_passes=False),
)
def _phase1(dst_hbm, node_hbm, eft_hbm, tgt_hbm, a_hbm, part_hbm,
            acc, nmap, tgtb, eftv, nob, msg, av, dstv, hid, wl,
            sem_d, sem_e, sem_g):
    c = lax.axis_index("c")
    s = lax.axis_index("s")
    zero16 = jnp.zeros((L,), jnp.float32)
    zero16i = jnp.zeros((L,), jnp.int32)
    iota = lax.iota(jnp.int32, L)
    tile_base = c * (NS * EPW) + s * EPW

    _build_map(tgt_hbm, tgtb, nmap)

    def hz(i, carry):
        hid[pl.ds(i * L, L)] = zero16i
        return carry
    lax.fori_loop(0, EB // L, hz, 0)

    def zrow(i, carry):
        for j in range(H):
            msg[i, pl.ds(j * L, L)] = zero16
        return carry
    lax.fori_loop(0, ACC_ROWS // NS, zrow, 0)
    pltpu.sync_copy(msg.at[pl.ds(0, ACC_ROWS // NS)],
                    acc.at[pl.ds(s * (ACC_ROWS // NS), ACC_ROWS // NS)])
    plsc.subcore_barrier()

    def _eft_slice(b):
        return eft_hbm.at[pl.ds(tile_base + b * EB, EB)]

    def _dst_slice(b):
        return dst_hbm.at[pl.ds(tile_base + b * EB, EB)]

    pltpu.sync_copy(_dst_slice(0), dstv.at[pl.ds(0, EB)])
    pltpu.async_copy(_eft_slice(0), eftv.at[pl.ds(0, EB)], sem_e)
    pltpu.async_copy(node_hbm.at[dstv.at[pl.ds(0, EB)]],
                     nob.at[pl.ds(0, EB)], sem_g)

    def block_body(blk, carry):
        p = blk & 1
        q = 1 - p
        base = tile_base + blk * EB
        nb = blk + 1

        eoff = p * EB
        noff = p * EB
        doff = p * EB

        @pl.when(nb < NBLK)
        def _():
            pltpu.async_copy(_dst_slice(nb),
                             dstv.at[pl.ds(q * EB, EB)], sem_d)

        pltpu.make_async_copy(_eft_slice(blk),
                              eftv.at[pl.ds(eoff, EB)], sem_e).wait()
        pltpu.make_async_copy(node_hbm.at[dstv.at[pl.ds(doff, EB)]],
                              nob.at[pl.ds(noff, EB)], sem_g).wait()

        def group_body(g, cnt):
            rows = g * L + iota
            rowse = rows + eoff
            rowsn = rows + noff

            def dot_body(t, carry):
                iotat, sims = carry
                col = iotat & (D - 1)
                new = []
                for h in range(H):
                    et = plsc.load_gather(eftv, [rowse, col])
                    nt = plsc.load_gather(nob, [rowsn, col])
                    new.append(sims[h] + et * nt)
                    if h < H - 1:
                        col = col + D
                return (iotat + 1, tuple(new))
            _, sims = lax.fori_loop(
                0, D, dot_body,
                (iota, tuple(jnp.zeros((L,), jnp.float32)
                             for _ in range(H))))
            sims = list(sims)
            m = sims[0]
            for h in range(1, H):
                m = jnp.maximum(m, sims[h])
            zs = [jnp.exp(sims[h] - m) for h in range(H)]
            ssum = (((zs[0] + zs[1]) + (zs[2] + zs[3]))
                    + ((zs[4] + zs[5]) + (zs[6] + zs[7])))
            rs = 1.0 / ssum
            a_hs = [zs[h] * rs for h in range(H)]
            hv = jnp.zeros((L,), jnp.int32)
            for h in range(H):
                plsc.store_scatter(av, [rows, hv], a_hs[h])
                if h < H - 1:
                    hv = hv + 1
            dvec = dstv[pl.ds(doff + g * L, L)]
            slots = plsc.load_gather(nmap, [dvec])
            hit = slots >= 0
            hi = hit.astype(jnp.int32)
            pos = cnt + plsc.cumsum(hi) - hi
            plsc.store_scatter(hid, [pos], rows, mask=hit)
            plsc.store_scatter(wl, [pos >> 6, pos & (CW - 1)], slots,
                               mask=hit)
            return cnt + jnp.sum(hi)
        cnt = lax.fori_loop(0, NG, group_body, jnp.int32(0))

        trash = jnp.full((L,), TRASH, jnp.int32)
        for t in range(CW // L):
            pp = cnt + t * L + iota
            plsc.store_scatter(wl, [pp >> 6, pp & (CW - 1)], trash)

        def hit_body(k, carry):
            drow = k * L + iota
            eids = plsc.load_gather(hid, [drow])
            eidse = eids + eoff
            hv = jnp.zeros((L,), jnp.int32)
            a_hs = []
            for h in range(H):
                a_hs.append(plsc.load_gather(av, [eids, hv]))
                if h < H - 1:
                    hv = hv + 1

            def msg_body(t, mc):
                iotat = mc
                col = iotat & (D - 1)
                for h in range(H):
                    et = plsc.load_gather(eftv, [eidse, col])
                    plsc.store_scatter(msg, [drow, col], et * a_hs[h])
                    if h < H - 1:
                        col = col + D
                return iotat + 1
            lax.fori_loop(0, D, msg_body, iota)
            return carry
        lax.fori_loop(0, (cnt + L - 1) >> 4, hit_body, 0)

        pltpu.sync_copy(av, a_hbm.at[pl.ds(base, EB)])
        nchunk = (cnt + CW - 1) >> 6

        def sc_body(j, carry):
            pltpu.sync_copy(msg.at[pl.ds(j * CW, CW)],
                            acc.at[wl.at[j]], add=True)
            return carry
        lax.fori_loop(0, nchunk, sc_body, 0)

        @pl.when(nb < NBLK)
        def _():
            pltpu.make_async_copy(_dst_slice(nb),
                                  dstv.at[pl.ds(q * EB, EB)], sem_d).wait()
            pltpu.async_copy(_eft_slice(nb),
                             eftv.at[pl.ds(q * EB, EB)], sem_e)
            pltpu.async_copy(node_hbm.at[dstv.at[pl.ds(q * EB, EB)]],
                             nob.at[pl.ds(q * EB, EB)], sem_g)
        return 0
    lax.fori_loop(0, NBLK, block_body, 0)

    plsc.subcore_barrier()
    r0 = s * (B_TGT // NS)
    pltpu.sync_copy(acc.at[pl.ds(r0, B_TGT // NS)],
                    msg.at[pl.ds(0, B_TGT // NS)])
    pltpu.sync_copy(msg.at[pl.ds(0, B_TGT // NS)],
                    part_hbm.at[c].at[pl.ds(r0, B_TGT // NS)])


@functools.partial(
    pl.kernel,
    out_type=jax.ShapeDtypeStruct((B_TGT, HD), jnp.float32),
    mesh=_mesh(),
    scratch_types=[
        pltpu.VMEM((N_NODES + L,), jnp.int32),
        pltpu.VMEM((B_TGT + L,), jnp.int32),
        pltpu.VMEM((TPW,), jnp.int32),
        pltpu.VMEM((TPW, HD), jnp.float32),
        pltpu.VMEM((TPW, HD), jnp.float32),
        pltpu.VMEM((TPW, HD), jnp.float32),
        pltpu.SemaphoreType.DMA,
    ],
    compiler_params=pltpu.CompilerParams(needs_layout_passes=False),
)
def _phase2(p0_hbm, p1_hbm, tgt_hbm, out_hbm,
            nmap, tgtb, slotv, r0v, r1v, ov, sem):
    c = lax.axis_index("c")
    s = lax.axis_index("s")
    base = (s * NC + c) * TPW

    _build_map(tgt_hbm, tgtb, nmap)
    for k in range(TPW // L):
        tv = tgtb[pl.ds(base + k * L, L)]
        slotv[pl.ds(k * L, L)] = plsc.load_gather(nmap, [tv])
    pltpu.async_copy(p0_hbm.at[slotv], r0v, sem).wait()
    pltpu.async_copy(p1_hbm.at[slotv], r1v, sem).wait()

    def row_body(i, carry):
        vs = []
        for j in range(H):
            vs.append(r0v[i, pl.ds(j * D, D)] + r1v[i, pl.ds(j * D, D)]
                      + 1e-15)
        ssq = vs[0] * vs[0]
        for j in range(1, H):
            ssq = ssq + vs[j] * vs[j]
        ib = plsc.bitcast(ssq, jnp.int32)
        y = plsc.bitcast(jnp.int32(0x5F3759DF) - (ib >> 1), jnp.float32)
        for _ in range(3):
            y = y * (1.5 - 0.5 * ssq * y * y)
        y = jnp.minimum(y, 1e12)
        for j in range(H):
            ov[i, pl.ds(j * D, D)] = vs[j] * y
        return carry
    lax.fori_loop(0, TPW, row_body, 0)
    pltpu.sync_copy(ov, out_hbm.at[pl.ds(base, TPW)])


def kernel(edge_index, node, eft, target_idx):
    dst = edge_index[1]
    node2 = node.reshape(N_NODES, HD)
    eft2 = eft.reshape(E_EDGES, HD)
    a2, parts = _phase1(dst, node2, eft2, target_idx)
    out2 = _phase2(parts[0], parts[1], target_idx)
    return (out2.reshape(B_TGT, H, D), a2.reshape(E_EDGES, H, 1))


# speedup vs baseline: 4.7993x; 1.3034x over previous
import functools

import jax
import jax.numpy as jnp
from jax import lax
from jax.experimental import pallas as pl
from jax.experimental.pallas import tpu as pltpu
from jax.experimental.pallas import tpu_sc as plsc

N_NODES = 10000
E_EDGES = 320000
H = 8
D = 16
HD = H * D
B_TGT = 1024

NC = 2
NS = 16
L = 16
NW = NC * NS
EPW = E_EDGES // NW
EB = 80
NBLK = EPW // EB
NG = EB // L
CW = 64
WLR = 3
MR = EB + CW
TRASH = B_TGT
ACC_ROWS = 1152
TPW = B_TGT // NW


def _mesh():
    return plsc.VectorSubcoreMesh(
        core_axis_name="c", subcore_axis_name="s",
        num_cores=NC, num_subcores=NS)


def _build_map(tgt_hbm, tgtb, nmap):
    pltpu.sync_copy(tgt_hbm, tgtb.at[pl.ds(0, B_TGT)])
    neg = jnp.full((L,), -1, jnp.int32)
    lane0 = lax.iota(jnp.int32, L) == 0

    def zbody(i, c):
        nmap[pl.ds(i * L, L)] = neg
        return c
    lax.fori_loop(0, N_NODES // L + 1, zbody, 0)

    def mbody(b, c):
        tv = tgtb[pl.ds(b, L)]
        plsc.store_scatter(nmap, [tv], jnp.full((L,), b, jnp.int32),
                           mask=lane0)
        return c
    lax.fori_loop(0, B_TGT, mbody, 0)


@functools.partial(
    pl.kernel,
    out_type=(
        jax.ShapeDtypeStruct((E_EDGES, H), jnp.float32),
        jax.ShapeDtypeStruct((NC, B_TGT, HD), jnp.float32),
    ),
    mesh=_mesh(),
    scratch_types=[
        pltpu.VMEM_SHARED((ACC_ROWS, HD), jnp.float32),
        pltpu.VMEM((N_NODES + L,), jnp.int32),
        pltpu.VMEM((B_TGT + L,), jnp.int32),
        pltpu.VMEM((2 * EB, HD), jnp.float32),
        pltpu.VMEM((2 * EB, HD), jnp.float32),
        pltpu.VMEM((MR, HD), jnp.float32),
        pltpu.VMEM((EB, H), jnp.float32),
        pltpu.VMEM((3 * EB,), jnp.int32),
        pltpu.VMEM((EB,), jnp.int32),
        pltpu.VMEM((WLR, CW), jnp.int32),
        pltpu.SemaphoreType.DMA,
        pltpu.SemaphoreType.DMA,
        pltpu.SemaphoreType.DMA,
    ],
    compiler_params=pltpu.CompilerParams(needs_layout_passes=False),
)
def _phase1(dst_hbm, node_hbm, eft_hbm, tgt_hbm, a_hbm, part_hbm,
            acc, nmap, tgtb, eftv, nob, msg, av, dstv, hid, wl,
            sem_d, sem_e, sem_g):
    c = lax.axis_index("c")
    s = lax.axis_index("s")
    zero16 = jnp.zeros((L,), jnp.float32)
    zero16i = jnp.zeros((L,), jnp.int32)
    iota = lax.iota(jnp.int32, L)
    tile_base = c * (NS * EPW) + s * EPW

    _build_map(tgt_hbm, tgtb, nmap)

    def hz(i, carry):
        hid[pl.ds(i * L, L)] = zero16i
        return carry
    lax.fori_loop(0, EB // L, hz, 0)

    def zrow(i, carry):
        for j in range(H):
            msg[i, pl.ds(j * L, L)] = zero16
        return carry
    lax.fori_loop(0, ACC_ROWS // NS, zrow, 0)
    pltpu.sync_copy(msg.at[pl.ds(0, ACC_ROWS // NS)],
                    acc.at[pl.ds(s * (ACC_ROWS // NS), ACC_ROWS // NS)])
    plsc.subcore_barrier()

    def _eft_slice(b):
        return eft_hbm.at[pl.ds(tile_base + b * EB, EB)]

    def _dst_slice(b):
        return dst_hbm.at[pl.ds(tile_base + b * EB, EB)]

    pltpu.sync_copy(_dst_slice(0), dstv.at[pl.ds(0, EB)])
    pltpu.async_copy(_eft_slice(0), eftv.at[pl.ds(0, EB)], sem_e)
    pltpu.async_copy(node_hbm.at[dstv.at[pl.ds(0, EB)]],
                     nob.at[pl.ds(0, EB)], sem_g)
    pltpu.async_copy(_dst_slice(1), dstv.at[pl.ds(EB, EB)], sem_d)

    def block_body(blk, carry):
        p = blk & 1
        q = 1 - p
        base = tile_base + blk * EB
        nb = blk + 1
        eoff = p * EB
        noff = p * EB
        doff = (blk % 3) * EB
        ndoff = (nb % 3) * EB

        pltpu.make_async_copy(_eft_slice(blk),
                              eftv.at[pl.ds(eoff, EB)], sem_e).wait()
        pltpu.make_async_copy(node_hbm.at[dstv.at[pl.ds(doff, EB)]],
                              nob.at[pl.ds(noff, EB)], sem_g).wait()

        @pl.when(nb < NBLK)
        def _():
            pltpu.make_async_copy(_dst_slice(nb),
                                  dstv.at[pl.ds(ndoff, EB)], sem_d).wait()
            pltpu.async_copy(_eft_slice(nb), eftv.at[pl.ds(q * EB, EB)],
                             sem_e)
            pltpu.async_copy(node_hbm.at[dstv.at[pl.ds(ndoff, EB)]],
                             nob.at[pl.ds(q * EB, EB)], sem_g)

        @pl.when(blk + 2 < NBLK)
        def _():
            pltpu.async_copy(_dst_slice(blk + 2),
                             dstv.at[pl.ds(((blk + 2) % 3) * EB, EB)],
                             sem_d)

        def group_body(g, cnt):
            rows = g * L + iota
            rowse = rows + eoff
            rowsn = rows + noff

            def dot_body(t, carry):
                iotat, sims = carry
                new = list(sims)
                for u in range(2):
                    col = (iotat + u) & (D - 1)
                    for h in range(H):
                        et = plsc.load_gather(eftv, [rowse, col])
                        nt = plsc.load_gather(nob, [rowsn, col])
                        new[h] = new[h] + et * nt
                        if h < H - 1:
                            col = col + D
                return (iotat + 2, tuple(new))
            _, sims = lax.fori_loop(
                0, D // 2, dot_body,
                (iota, tuple(jnp.zeros((L,), jnp.float32)
                             for _ in range(H))))
            sims = list(sims)
            zs = [jnp.exp(sims[h]) for h in range(H)]
            ssum = (((zs[0] + zs[1]) + (zs[2] + zs[3]))
                    + ((zs[4] + zs[5]) + (zs[6] + zs[7])))
            rs = 1.0 / ssum
            a_hs = [zs[h] * rs for h in range(H)]
            hv = jnp.zeros((L,), jnp.int32)
            for h in range(H):
                plsc.store_scatter(av, [rows, hv], a_hs[h])
                if h < H - 1:
                    hv = hv + 1
            dvec = dstv[pl.ds(doff + g * L, L)]
            slots = plsc.load_gather(nmap, [dvec])
            hit = slots >= 0
            hi = hit.astype(jnp.int32)
            pos = cnt + plsc.cumsum(hi) - hi
            plsc.store_scatter(hid, [pos], rows, mask=hit)
            plsc.store_scatter(wl, [pos >> 6, pos & (CW - 1)], slots,
                               mask=hit)
            return cnt + jnp.sum(hi)
        cnt = lax.fori_loop(0, NG, group_body, jnp.int32(0))

        trash = jnp.full((L,), TRASH, jnp.int32)
        for t in range(CW // L):
            pp = cnt + t * L + iota
            plsc.store_scatter(wl, [pp >> 6, pp & (CW - 1)], trash)

        def hit_body(k, carry):
            drow = k * L + iota
            eids = plsc.load_gather(hid, [drow])
            eidse = eids + eoff
            hv = jnp.zeros((L,), jnp.int32)
            a_hs = []
            for h in range(H):
                a_hs.append(plsc.load_gather(av, [eids, hv]))
                if h < H - 1:
                    hv = hv + 1

            def msg_body(t, mc):
                iotat = mc
                col = iotat & (D - 1)
                for h in range(H):
                    et = plsc.load_gather(eftv, [eidse, col])
                    plsc.store_scatter(msg, [drow, col], et * a_hs[h])
                    if h < H - 1:
                        col = col + D
                return iotat + 1
            lax.fori_loop(0, D, msg_body, iota)
            return carry
        lax.fori_loop(0, (cnt + L - 1) >> 4, hit_body, 0)

        pltpu.sync_copy(av, a_hbm.at[pl.ds(base, EB)])
        nchunk = (cnt + CW - 1) >> 6

        def sc_body(j, carry):
            pltpu.sync_copy(msg.at[pl.ds(j * CW, CW)],
                            acc.at[wl.at[j]], add=True)
            return carry
        lax.fori_loop(0, nchunk, sc_body, 0)

        return 0
    lax.fori_loop(0, NBLK, block_body, 0)

    plsc.subcore_barrier()
    r0 = s * (B_TGT // NS)
    pltpu.sync_copy(acc.at[pl.ds(r0, B_TGT // NS)],
                    msg.at[pl.ds(0, B_TGT // NS)])
    pltpu.sync_copy(msg.at[pl.ds(0, B_TGT // NS)],
                    part_hbm.at[c].at[pl.ds(r0, B_TGT // NS)])


@functools.partial(
    pl.kernel,
    out_type=jax.ShapeDtypeStruct((B_TGT, HD), jnp.float32),
    mesh=_mesh(),
    scratch_types=[
        pltpu.VMEM((N_NODES + L,), jnp.int32),
        pltpu.VMEM((B_TGT + L,), jnp.int32),
        pltpu.VMEM((TPW,), jnp.int32),
        pltpu.VMEM((TPW, HD), jnp.float32),
        pltpu.VMEM((TPW, HD), jnp.float32),
        pltpu.VMEM((TPW, HD), jnp.float32),
        pltpu.SemaphoreType.DMA,
    ],
    compiler_params=pltpu.CompilerParams(needs_layout_passes=False),
)
def _phase2(p0_hbm, p1_hbm, tgt_hbm, out_hbm,
            nmap, tgtb, slotv, r0v, r1v, ov, sem):
    c = lax.axis_index("c")
    s = lax.axis_index("s")
    base = (s * NC + c) * TPW

    _build_map(tgt_hbm, tgtb, nmap)
    for k in range(TPW // L):
        tv = tgtb[pl.ds(base + k * L, L)]
        slotv[pl.ds(k * L, L)] = plsc.load_gather(nmap, [tv])
    pltpu.async_copy(p0_hbm.at[slotv], r0v, sem).wait()
    pltpu.async_copy(p1_hbm.at[slotv], r1v, sem).wait()

    def row_body(i, carry):
        vs = []
        for j in range(H):
            vs.append(r0v[i, pl.ds(j * D, D)] + r1v[i, pl.ds(j * D, D)]
                      + 1e-15)
        ssq = vs[0] * vs[0]
        for j in range(1, H):
            ssq = ssq + vs[j] * vs[j]
        ib = plsc.bitcast(ssq, jnp.int32)
        y = plsc.bitcast(jnp.int32(0x5F3759DF) - (ib >> 1), jnp.float32)
        for _ in range(3):
            y = y * (1.5 - 0.5 * ssq * y * y)
        y = jnp.minimum(y, 1e12)
        for j in range(H):
            ov[i, pl.ds(j * D, D)] = vs[j] * y
        return carry
    lax.fori_loop(0, TPW, row_body, 0)
    pltpu.sync_copy(ov, out_hbm.at[pl.ds(base, TPW)])


def kernel(edge_index, node, eft, target_idx):
    dst = edge_index[1]
    node2 = node.reshape(N_NODES, HD)
    eft2 = eft.reshape(E_EDGES, HD)
    a2, parts = _phase1(dst, node2, eft2, target_idx)
    out2 = _phase2(parts[0], parts[1], target_idx)
    return (out2.reshape(B_TGT, H, D), a2.reshape(E_EDGES, H, 1))
